# Initial kernel scaffold; baseline (speedup 1.0000x reference)
import jax
import jax.numpy as jnp
from jax.experimental import pallas as pl


def kernel(input_ids, weight):
    raise NotImplementedError("write your pallas kernel here")


# speedup vs baseline: 1.3186x; 1.3186x over previous
import functools

import jax
import jax.numpy as jnp
from jax import lax
from jax.experimental import pallas as pl
from jax.experimental.pallas import tpu as pltpu
from jax.experimental.pallas import tpu_sc as plsc

_NC = 2
_NS = 16
_NW = _NC * _NS
_C = 16
_NBUF = 2


@functools.lru_cache(maxsize=None)
def _make_sc_gather(B, V, D):
    b_per_w = B // _NW
    nchunk = b_per_w // _C
    assert b_per_w % _C == 0 and nchunk % _NBUF == 0
    mesh = plsc.VectorSubcoreMesh(core_axis_name="c", subcore_axis_name="s")

    @functools.partial(
        pl.kernel,
        out_type=jax.ShapeDtypeStruct((B, D), jnp.float32),
        mesh=mesh,
        scratch_types=[
            pltpu.VMEM((nchunk, _C), jnp.int32),
            pltpu.VMEM((_NBUF, _C, D), jnp.float32),
            pltpu.SemaphoreType.DMA((_NBUF,)),
        ],
    )
    def sc_gather(idx_hbm, table_hbm, out_hbm, idx_v, bufs, sems):
        wid = lax.axis_index("s") * _NC + lax.axis_index("c")
        base = wid * b_per_w
        pltpu.sync_copy(idx_hbm.at[wid], idx_v)
        for b in range(_NBUF):
            pltpu.async_copy(table_hbm.at[idx_v.at[b]], bufs.at[b], sems.at[b])

        def step(i, carry):
            for b in range(_NBUF):
                g = i * _NBUF + b
                pltpu.make_async_copy(
                    table_hbm.at[idx_v.at[g]], bufs.at[b], sems.at[b]
                ).wait()
                pltpu.sync_copy(bufs.at[b], out_hbm.at[pl.ds(base + g * _C, _C)])
                ng = g + _NBUF

                @pl.when(ng < nchunk)
                def _start_next():
                    pltpu.async_copy(
                        table_hbm.at[idx_v.at[ng]], bufs.at[b], sems.at[b]
                    )

            return carry

        lax.fori_loop(0, nchunk // _NBUF, step, 0)

    return sc_gather


def kernel(input_ids, weight):
    B = input_ids.shape[0] * input_ids.shape[1]
    V, D = weight.shape
    idx = input_ids.reshape(_NW, B // (_NW * _C), _C)
    out = _make_sc_gather(B, V, D)(idx, weight)
    return out.reshape(input_ids.shape[0], input_ids.shape[1], D)


# speedup vs baseline: 1.3320x; 1.0102x over previous
import functools

import jax
import jax.numpy as jnp
from jax import lax
from jax.experimental import pallas as pl
from jax.experimental.pallas import tpu as pltpu
from jax.experimental.pallas import tpu_sc as plsc

_NC = 2
_NS = 16
_NW = _NC * _NS
_C = 16
_NBUF = 3
_SKEW = 2


@functools.lru_cache(maxsize=None)
def _make_sc_gather(B, V, D):
    b_per_w = B // _NW
    nchunk = b_per_w // _C
    assert b_per_w % _C == 0
    nslot = nchunk + _SKEW
    niter = -(-nslot // _NBUF)
    mesh = plsc.VectorSubcoreMesh(core_axis_name="c", subcore_axis_name="s")

    @functools.partial(
        pl.kernel,
        out_type=jax.ShapeDtypeStruct((B, D), jnp.float32),
        mesh=mesh,
        scratch_types=[
            pltpu.VMEM((nchunk, _C), jnp.int32),
            pltpu.VMEM((_NBUF, _C, D), jnp.float32),
            pltpu.SemaphoreType.DMA((_NBUF,)),
            pltpu.SemaphoreType.DMA((_NBUF,)),
        ],
    )
    def sc_gather(idx_hbm, table_hbm, out_hbm, idx_v, bufs, gsems, wsems):
        wid = lax.axis_index("s") * _NC + lax.axis_index("c")
        base = wid * b_per_w
        pltpu.sync_copy(idx_hbm.at[wid], idx_v)

        def step(i, carry):
            for b in range(_NBUF):
                g = i * _NBUF + b
                bw = (b + _NBUF - _SKEW) % _NBUF

                @pl.when(jnp.logical_and(g >= _SKEW, g < nchunk + _SKEW))
                def _write():
                    wg = g - _SKEW
                    pltpu.make_async_copy(
                        table_hbm.at[idx_v.at[wg]], bufs.at[bw], gsems.at[bw]
                    ).wait()
                    pltpu.async_copy(
                        bufs.at[bw],
                        out_hbm.at[pl.ds(base + wg * _C, _C)],
                        wsems.at[bw],
                    )

                @pl.when(g < nchunk)
                def _gather():
                    @pl.when(g >= _NBUF)
                    def _drain():
                        pg = g - _NBUF
                        pltpu.make_async_copy(
                            bufs.at[b],
                            out_hbm.at[pl.ds(base + pg * _C, _C)],
                            wsems.at[b],
                        ).wait()

                    pltpu.async_copy(
                        table_hbm.at[idx_v.at[g]], bufs.at[b], gsems.at[b]
                    )

            return carry

        lax.fori_loop(0, niter, step, 0)

    return sc_gather


def kernel(input_ids, weight):
    B = input_ids.shape[0] * input_ids.shape[1]
    V, D = weight.shape
    idx = input_ids.reshape(_NW, B // (_NW * _C), _C)
    out = _make_sc_gather(B, V, D)(idx, weight)
    return out.reshape(input_ids.shape[0], input_ids.shape[1], D)
